# baseline (speedup 1.0000x reference)
import functools

import jax
import jax.numpy as jnp
from jax import lax
from jax.experimental import pallas as pl
from jax.experimental.pallas import tpu as pltpu
from jax.experimental.pallas import tpu_sc as plsc

WINDOW_SIZE = 16
TEMPERATURE = 0.01

_N_WORKERS = 32
_N_SC_MAPS = 192
_TC_BLOCK = 32


def _tc_kernel(hm_ref, out_ref):
    K, H, W = hm_ref.shape
    half = WINDOW_SIZE // 2
    BAND = WINDOW_SIZE + 8

    hm = hm_ref[...]

    m_v = jnp.max(hm, axis=(1, 2))
    rows = lax.broadcasted_iota(jnp.int32, (K, H, W), 1)
    cols = lax.broadcasted_iota(jnp.int32, (K, H, W), 2)
    flat = rows * W + cols
    idx_v = jnp.min(
        jnp.where(hm == m_v[:, None, None], flat, jnp.int32(H * W)),
        axis=(1, 2),
    )
    y0_v = idx_v // W
    x0_v = idx_v - y0_v * W

    xmin_v = jnp.maximum(x0_v - half, 0)
    xmax_v = jnp.minimum(x0_v + half, W)
    ymin_v = jnp.maximum(y0_v - half, 0)
    ymax_v = jnp.minimum(y0_v + half, H)

    ystart_v = jnp.minimum((jnp.maximum(y0_v - half, 0) // 8) * 8, H - BAND)

    bands = jnp.stack(
        [
            hm_ref[j, pl.ds(pl.multiple_of(ystart_v[j], 8), BAND), :]
            for j in range(K)
        ]
    )

    bgy = lax.broadcasted_iota(jnp.int32, (K, BAND, W), 1)
    bgx = lax.broadcasted_iota(jnp.int32, (K, BAND, W), 2)
    gy = ystart_v[:, None, None] + bgy
    mask = (
        (bgx >= xmin_v[:, None, None])
        & (bgx < xmax_v[:, None, None])
        & (gy >= ymin_v[:, None, None])
        & (gy < ymax_v[:, None, None])
    )

    e = jnp.where(
        mask, jnp.exp((bands - m_v[:, None, None]) * (1.0 / TEMPERATURE)), 0.0
    )
    s = jnp.sum(e, axis=(1, 2))
    p = e / s[:, None, None]

    fx = bgx.astype(jnp.float32)
    fy = gy.astype(jnp.float32)
    x_mean = jnp.sum(fx * p, axis=(1, 2))
    y_mean = jnp.sum(fy * p, axis=(1, 2))
    dx = fx - x_mean[:, None, None]
    dy = fy - y_mean[:, None, None]
    var_xx = jnp.sum(p * dx * dx, axis=(1, 2))
    var_yy = jnp.sum(p * dy * dy, axis=(1, 2))
    cov_xy = jnp.sum(p * dx * dy, axis=(1, 2))

    out_ref[...] = jnp.stack(
        [
            x_mean * (1.0 / (W - 1)),
            y_mean * (1.0 / (H - 1)),
            var_xx,
            cov_xy,
            cov_xy,
            var_yy,
            var_xx + var_yy,
            jnp.zeros((K,), jnp.float32),
        ],
        axis=-1,
    )


def _tc_softargmax(hm, n_tc):
    H, W = hm.shape[1], hm.shape[2]
    K = _TC_BLOCK
    return pl.pallas_call(
        _tc_kernel,
        grid=(n_tc // K,),
        in_specs=[pl.BlockSpec((K, H, W), lambda i: (i, 0, 0))],
        out_specs=pl.BlockSpec((K, 8), lambda i: (i, 0)),
        out_shape=jax.ShapeDtypeStruct((n_tc, 8), jnp.float32),
    )(hm)


def _lane_reduce(v, binop, lane):
    for d in (8, 4, 2, 1):
        w = v.at[lane ^ d].get(mode="promise_in_bounds")
        v = binop(v, w)
    return v


_NB = 4
_CH_ROWS = 32
_N_CHUNKS = 256 // _CH_ROWS


def _sc_body(n_sc, base_map, hm_hbm, out_hbm, ring, row_vmem, win_vmem,
             res_vmem, sem0, sem1, sem2, sem3, semw):
    H, W = 256, 256
    half = WINDOW_SIZE // 2
    mpw = n_sc // _N_WORKERS
    cid = lax.axis_index("c")
    sid = lax.axis_index("s")
    wid = sid * 2 + cid
    base0 = base_map + wid * mpw
    sems = [sem0, sem1, sem2, sem3]
    total_chunks = mpw * _N_CHUNKS

    lane = lax.iota(jnp.int32, 16)
    big_f = jnp.full((16,), -3.0e38, jnp.float32)
    big_i = jnp.full((16,), jnp.int32(H * W), jnp.int32)
    zero = jnp.zeros((16,), jnp.float32)
    zero_i = jnp.zeros((16,), jnp.int32)

    def chunk_copy(g, b):
        off = (base0 + g // _N_CHUNKS) * (H * W) \
            + (g % _N_CHUNKS) * (_CH_ROWS * W)
        return pltpu.make_async_copy(
            hm_hbm.at[pl.ds(pl.multiple_of(off, 8), _CH_ROWS * W)],
            ring.at[b],
            sems[b],
        )

    for g in range(_NB - 1):
        chunk_copy(g, g).start()

    def finalize(g, runv, rowv):
        mbase = (base0 + g // _N_CHUNKS) * (H * W)
        mvec = _lane_reduce(runv, jnp.maximum, lane)
        rstar_v = _lane_reduce(
            jnp.where(runv == mvec, rowv, jnp.int32(H)), jnp.minimum, lane
        )
        y0 = rstar_v[0]
        pltpu.sync_copy(
            hm_hbm.at[pl.ds(pl.multiple_of(mbase + y0 * W, 8), W)], row_vmem
        )
        colv = big_i
        for j in range(W // 16):
            e = row_vmem[pl.ds(j * 16, 16)]
            colv = jnp.where((e == mvec) & (colv == big_i),
                             jnp.int32(j * 16) + lane, colv)
        x0 = _lane_reduce(colv, jnp.minimum, lane)[0]

        xmin = jnp.maximum(x0 - half, 0)
        xmax = jnp.minimum(x0 + half, W)
        ymin = jnp.maximum(y0 - half, 0)
        ymax = jnp.minimum(y0 + half, H)
        xs = jnp.clip(x0 - half, 0, W - 16)
        base = jnp.minimum((xs // 16) * 16, W - 32)
        off = xs - base
        sh = lane + off
        shidx = sh & 15
        lowsel = sh < 16

        basea = pl.multiple_of(base, 16)
        wcopies = []
        for i in range(16):
            yw = jnp.minimum(ymin + i, H - 1)
            cp = pltpu.make_async_copy(
                hm_hbm.at[pl.ds(pl.multiple_of(mbase + yw * W + basea, 16),
                                32)],
                win_vmem.at[i],
                semw,
            )
            cp.start()
            wcopies.append(cp)
        for cp in wcopies:
            cp.wait()

        gx = xs + lane
        maskx = (gx >= xmin) & (gx < xmax)
        fxc = (gx - x0).astype(jnp.float32)
        fxc2 = fxc * fxc

        se = sex = sey = sexx = seyy = sexy = zero
        for i in range(16):
            y = ymin + i
            v0 = win_vmem[i, pl.ds(0, 16)]
            v1 = win_vmem[i, pl.ds(16, 16)]
            row = jnp.where(
                lowsel,
                v0.at[shidx].get(mode="promise_in_bounds"),
                v1.at[shidx].get(mode="promise_in_bounds"),
            )
            wrow = jnp.where(y < ymax, 1.0, 0.0)
            e = jnp.where(maskx, jnp.exp((row - mvec) * (1.0 / TEMPERATURE)),
                          zero) * wrow
            fy = (y - y0).astype(jnp.float32)
            ex = e * fxc
            se = se + e
            sex = sex + ex
            sey = sey + e * fy
            sexx = sexx + e * fxc2
            seyy = seyy + e * (fy * fy)
            sexy = sexy + ex * fy

        add = jnp.add
        s0 = _lane_reduce(se, add, lane)
        mx = _lane_reduce(sex, add, lane) / s0
        my = _lane_reduce(sey, add, lane) / s0
        var_xx = _lane_reduce(sexx, add, lane) / s0 - mx * mx
        var_yy = _lane_reduce(seyy, add, lane) / s0 - my * my
        cov_xy = _lane_reduce(sexy, add, lane) / s0 - mx * my
        x_mean = x0.astype(jnp.float32) + mx
        y_mean = y0.astype(jnp.float32) + my

        vals = jnp.where(
            lane == 0,
            x_mean * (1.0 / (W - 1)),
            jnp.where(
                lane == 1,
                y_mean * (1.0 / (H - 1)),
                jnp.where(
                    lane == 2,
                    var_xx,
                    jnp.where(
                        (lane == 3) | (lane == 4),
                        cov_xy,
                        jnp.where(
                            lane == 5,
                            var_yy,
                            jnp.where(lane == 6, var_xx + var_yy, 0.0),
                        ),
                    ),
                ),
            ),
        )
        ml = g // _N_CHUNKS
        res_vmem[pl.ds(pl.multiple_of(ml * 16, 16), 16)] = vals

    def map_body(t, dummy):
        runv, rowv = big_f, zero_i
        for j in range(_N_CHUNKS):
            g = t * _N_CHUNKS + j
            chunk_copy(g, j % _NB).wait()
            r0 = j * _CH_ROWS
            b = j % _NB

            def scan_row(i, carry):
                rv, ro = carry
                rowm = ring[b, pl.ds(i * 256, 16)]
                for jj in range(1, W // 16):
                    rowm = jnp.maximum(
                        rowm, ring[b, pl.ds(i * 256 + jj * 16, 16)])
                gt = rowm > rv
                rv = jnp.where(gt, rowm, rv)
                ro = jnp.where(gt, r0 + i, ro)
                return rv, ro

            runv, rowv = lax.fori_loop(0, _CH_ROWS, scan_row, (runv, rowv))
            gn = g + _NB - 1
            @pl.when(gn < total_chunks)
            def _():
                chunk_copy(gn, (j + _NB - 1) % _NB).start()
        finalize(t * _N_CHUNKS + _N_CHUNKS - 1, runv, rowv)
        return dummy

    lax.fori_loop(0, mpw, map_body, 0)

    pltpu.sync_copy(res_vmem, out_hbm.at[pl.ds(wid * (mpw * 16), mpw * 16)])


def _sc_softargmax(hm, n_sc, base_map):
    mpw = n_sc // _N_WORKERS
    mesh = plsc.VectorSubcoreMesh(core_axis_name="c", subcore_axis_name="s")
    fn = functools.partial(
        pl.kernel,
        mesh=mesh,
        out_type=jax.ShapeDtypeStruct((n_sc * 16,), jnp.float32),
        scratch_types=[
            pltpu.VMEM((_NB, _CH_ROWS * 256), jnp.float32),
            pltpu.VMEM((256,), jnp.float32),
            pltpu.VMEM((16, 32), jnp.float32),
            pltpu.VMEM((mpw * 16,), jnp.float32),
            pltpu.SemaphoreType.DMA,
            pltpu.SemaphoreType.DMA,
            pltpu.SemaphoreType.DMA,
            pltpu.SemaphoreType.DMA,
            pltpu.SemaphoreType.DMA,
        ],
    )(functools.partial(_sc_body, n_sc, base_map))
    return fn(hm)


def kernel(heatmap):
    B, C, H, W = heatmap.shape
    n = B * C
    hm = heatmap.reshape(n, H, W)

    n_sc = _N_SC_MAPS
    n_tc = n - n_sc

    sc_stats = _sc_softargmax(
        hm.reshape(n * H * W), n_sc, n_tc
    ).reshape(n_sc, 16)[:, :8]
    tc_stats = _tc_softargmax(hm, n_tc)

    stats = jnp.concatenate([tc_stats, sc_stats], axis=0)
    coords = stats[:, 0:2].reshape(B, C, 2)
    cov = stats[:, 2:6].reshape(B, C, 2, 2)
    spread = stats[:, 6:7].reshape(B, C, 1)
    return (coords, cov, spread)


# speedup vs baseline: 2.4461x; 2.4461x over previous
import functools

import jax
import jax.numpy as jnp
from jax import lax
from jax.experimental import pallas as pl
from jax.experimental.pallas import tpu as pltpu
from jax.experimental.pallas import tpu_sc as plsc

WINDOW_SIZE = 16
TEMPERATURE = 0.01

_N_WORKERS = 32
_N_SC_MAPS = 160
_TC_BLOCK = 32


def _tc_kernel(hm_ref, out_ref):
    K, H, W = hm_ref.shape
    half = WINDOW_SIZE // 2
    BAND = WINDOW_SIZE + 8

    hm = hm_ref[...]

    m_v = jnp.max(hm, axis=(1, 2))
    rows = lax.broadcasted_iota(jnp.int32, (K, H, W), 1)
    cols = lax.broadcasted_iota(jnp.int32, (K, H, W), 2)
    flat = rows * W + cols
    idx_v = jnp.min(
        jnp.where(hm == m_v[:, None, None], flat, jnp.int32(H * W)),
        axis=(1, 2),
    )
    y0_v = idx_v // W
    x0_v = idx_v - y0_v * W

    xmin_v = jnp.maximum(x0_v - half, 0)
    xmax_v = jnp.minimum(x0_v + half, W)
    ymin_v = jnp.maximum(y0_v - half, 0)
    ymax_v = jnp.minimum(y0_v + half, H)

    ystart_v = jnp.minimum((jnp.maximum(y0_v - half, 0) // 8) * 8, H - BAND)

    bands = jnp.stack(
        [
            hm_ref[j, pl.ds(pl.multiple_of(ystart_v[j], 8), BAND), :]
            for j in range(K)
        ]
    )

    bgy = lax.broadcasted_iota(jnp.int32, (K, BAND, W), 1)
    bgx = lax.broadcasted_iota(jnp.int32, (K, BAND, W), 2)
    gy = ystart_v[:, None, None] + bgy
    mask = (
        (bgx >= xmin_v[:, None, None])
        & (bgx < xmax_v[:, None, None])
        & (gy >= ymin_v[:, None, None])
        & (gy < ymax_v[:, None, None])
    )

    e = jnp.where(
        mask, jnp.exp((bands - m_v[:, None, None]) * (1.0 / TEMPERATURE)), 0.0
    )
    s = jnp.sum(e, axis=(1, 2))
    p = e / s[:, None, None]

    fx = bgx.astype(jnp.float32)
    fy = gy.astype(jnp.float32)
    x_mean = jnp.sum(fx * p, axis=(1, 2))
    y_mean = jnp.sum(fy * p, axis=(1, 2))
    dx = fx - x_mean[:, None, None]
    dy = fy - y_mean[:, None, None]
    var_xx = jnp.sum(p * dx * dx, axis=(1, 2))
    var_yy = jnp.sum(p * dy * dy, axis=(1, 2))
    cov_xy = jnp.sum(p * dx * dy, axis=(1, 2))

    out_ref[...] = jnp.stack(
        [
            x_mean * (1.0 / (W - 1)),
            y_mean * (1.0 / (H - 1)),
            var_xx,
            cov_xy,
            cov_xy,
            var_yy,
            var_xx + var_yy,
            jnp.zeros((K,), jnp.float32),
        ],
        axis=-1,
    )


def _tc_softargmax(hm, n_tc):
    H, W = hm.shape[1], hm.shape[2]
    K = _TC_BLOCK
    return pl.pallas_call(
        _tc_kernel,
        grid=(n_tc // K,),
        in_specs=[pl.BlockSpec((K, H, W), lambda i: (i, 0, 0))],
        out_specs=pl.BlockSpec((K, 8), lambda i: (i, 0)),
        out_shape=jax.ShapeDtypeStruct((n_tc, 8), jnp.float32),
    )(hm)


def _lane_reduce(v, binop, lane):
    for d in (8, 4, 2, 1):
        w = v.at[lane ^ d].get(mode="promise_in_bounds")
        v = binop(v, w)
    return v


def _sc_body(n_sc, base_map, hm_hbm, out_hbm, map_vmem, res_vmem):
    H, W = 256, 256
    half = WINDOW_SIZE // 2
    mpw = n_sc // _N_WORKERS
    cid = lax.axis_index("c")
    sid = lax.axis_index("s")
    wid = sid * 2 + cid

    lane = lax.iota(jnp.int32, 16)
    big_f = jnp.full((16,), -3.0e38, jnp.float32)
    big_i = jnp.full((16,), jnp.int32(H * W), jnp.int32)

    for t in range(mpw):
        mid = base_map + wid * mpw + t
        pltpu.sync_copy(hm_hbm.at[mid], map_vmem)

        def scan_row(r, carry):
            runv, rowv = carry
            rowm = map_vmem[r, pl.ds(0, 16)]
            for j in range(1, W // 16):
                rowm = jnp.maximum(rowm, map_vmem[r, pl.ds(j * 16, 16)])
            gt = rowm > runv
            runv = jnp.where(gt, rowm, runv)
            rowv = jnp.where(gt, r, rowv)
            return runv, rowv

        runv, rowv = lax.fori_loop(
            0, H, scan_row, (big_f, jnp.zeros((16,), jnp.int32))
        )
        mvec = _lane_reduce(runv, jnp.maximum, lane)
        rstar_v = _lane_reduce(
            jnp.where(runv == mvec, rowv, jnp.int32(H)), jnp.minimum, lane
        )
        y0 = rstar_v[0]
        def col_scan(carry_j):
            colv = big_i
            for j in range(W // 16):
                e = map_vmem[y0, pl.ds(j * 16, 16)]
                colv = jnp.where((e == mvec) & (colv == big_i),
                                 jnp.int32(j * 16) + lane, colv)
            return colv
        colv = col_scan(None)
        x0 = _lane_reduce(colv, jnp.minimum, lane)[0]
        idx = y0 * W + x0

        xmin = jnp.maximum(x0 - half, 0)
        xmax = jnp.minimum(x0 + half, W)
        ymin = jnp.maximum(y0 - half, 0)
        ymax = jnp.minimum(y0 + half, H)
        xs = jnp.clip(x0 - half, 0, W - 16)
        base = jnp.minimum((xs // 16) * 16, W - 32)
        off = xs - base
        sh = lane + off
        shidx = sh & 15
        lowsel = sh < 16

        gx = xs + lane
        maskx = (gx >= xmin) & (gx < xmax)
        fxc = (gx - x0).astype(jnp.float32)
        fxc2 = fxc * fxc
        zero = jnp.zeros((16,), jnp.float32)

        def win_row(y, carry):
            se, sex, sey, sexx, seyy, sexy = carry
            v0 = map_vmem[y, pl.ds(pl.multiple_of(base, 16), 16)]
            v1 = map_vmem[y, pl.ds(pl.multiple_of(base + 16, 16), 16)]
            row = jnp.where(
                lowsel,
                v0.at[shidx].get(mode="promise_in_bounds"),
                v1.at[shidx].get(mode="promise_in_bounds"),
            )
            e = jnp.where(maskx, jnp.exp((row - mvec) * (1.0 / TEMPERATURE)),
                          zero)
            fy = (y - y0).astype(jnp.float32)
            ex = e * fxc
            se = se + e
            sex = sex + ex
            sey = sey + e * fy
            sexx = sexx + e * fxc2
            seyy = seyy + e * (fy * fy)
            sexy = sexy + ex * fy
            return se, sex, sey, sexx, seyy, sexy

        se, sex, sey, sexx, seyy, sexy = lax.fori_loop(
            ymin, ymax, win_row, (zero, zero, zero, zero, zero, zero)
        )
        add = jnp.add
        s0 = _lane_reduce(se, add, lane)
        mx = _lane_reduce(sex, add, lane) / s0
        my = _lane_reduce(sey, add, lane) / s0
        var_xx = _lane_reduce(sexx, add, lane) / s0 - mx * mx
        var_yy = _lane_reduce(seyy, add, lane) / s0 - my * my
        cov_xy = _lane_reduce(sexy, add, lane) / s0 - mx * my
        x_mean = x0.astype(jnp.float32) + mx
        y_mean = y0.astype(jnp.float32) + my

        vals = jnp.where(
            lane == 0,
            x_mean * (1.0 / (W - 1)),
            jnp.where(
                lane == 1,
                y_mean * (1.0 / (H - 1)),
                jnp.where(
                    lane == 2,
                    var_xx,
                    jnp.where(
                        (lane == 3) | (lane == 4),
                        cov_xy,
                        jnp.where(
                            lane == 5,
                            var_yy,
                            jnp.where(lane == 6, var_xx + var_yy, 0.0),
                        ),
                    ),
                ),
            ),
        )
        res_vmem[pl.ds(t * 16, 16)] = vals

    pltpu.sync_copy(res_vmem, out_hbm.at[pl.ds(wid * (mpw * 16), mpw * 16)])


def _sc_softargmax(hm, n_sc, base_map):
    mpw = n_sc // _N_WORKERS
    mesh = plsc.VectorSubcoreMesh(core_axis_name="c", subcore_axis_name="s")
    fn = functools.partial(
        pl.kernel,
        mesh=mesh,
        out_type=jax.ShapeDtypeStruct((n_sc * 16,), jnp.float32),
        scratch_types=[
            pltpu.VMEM((256, 256), jnp.float32),
            pltpu.VMEM((mpw * 16,), jnp.float32),
        ],
    )(functools.partial(_sc_body, n_sc, base_map))
    return fn(hm)


def kernel(heatmap):
    B, C, H, W = heatmap.shape
    n = B * C
    hm = heatmap.reshape(n, H, W)

    n_sc = _N_SC_MAPS
    n_tc = n - n_sc

    sc_stats = _sc_softargmax(hm, n_sc, n_tc).reshape(n_sc, 16)[:, :8]
    tc_stats = _tc_softargmax(hm, n_tc)

    stats = jnp.concatenate([tc_stats, sc_stats], axis=0)
    coords = stats[:, 0:2].reshape(B, C, 2)
    cov = stats[:, 2:6].reshape(B, C, 2, 2)
    spread = stats[:, 6:7].reshape(B, C, 1)
    return (coords, cov, spread)
